# baseline (speedup 1.0000x reference)
import functools

import jax
import jax.numpy as jnp
from jax import lax
from jax.experimental import pallas as pl
from jax.experimental.pallas import tpu as pltpu
from jax.experimental.pallas import tpu_sc as plsc

_B, _S, _D = 4, 8192, 4096
_N = _B * _S
_NC, _NS = 2, 16
_NW = _NC * _NS
_PER_W = _N // _NW
_CHUNK = 128
_NCHUNK = _PER_W // _CHUNK

_mesh = plsc.VectorSubcoreMesh(core_axis_name="c", subcore_axis_name="s")


@functools.partial(
    pl.kernel,
    mesh=_mesh,
    out_type=jax.ShapeDtypeStruct((_NW, _NCHUNK, _CHUNK), jnp.float32),
    scratch_types=[
        pltpu.VMEM((_NCHUNK, _CHUNK), jnp.int32),
        pltpu.VMEM((_NCHUNK, _CHUNK), jnp.float32),
        pltpu.SemaphoreType.DMA,
    ],
)
def _select_gather(flat_hbm, idx_hbm, out_hbm, idx_v, vals_v, sem):
    wid = lax.axis_index("s") * _NC + lax.axis_index("c")
    pltpu.sync_copy(idx_hbm.at[wid], idx_v)
    copies = [
        pltpu.make_async_copy(flat_hbm.at[idx_v.at[j]], vals_v.at[j], sem)
        for j in range(_NCHUNK)
    ]
    for c in copies:
        c.start()
    for c in copies:
        c.wait()
    pltpu.sync_copy(vals_v, out_hbm.at[wid])


def kernel(input_):
    flat = input_.reshape(_N * _D)
    idx = (jnp.arange(_N, dtype=jnp.int32) * _D + 3).reshape(_NW, _NCHUNK, _CHUNK)
    out = _select_gather(flat, idx)
    return out.reshape(_B, _S, 1)


# speedup vs baseline: 14.1876x; 14.1876x over previous
import functools

import jax
import jax.numpy as jnp
from jax import lax
from jax.experimental import pallas as pl
from jax.experimental.pallas import tpu as pltpu
from jax.experimental.pallas import tpu_sc as plsc

_B, _S, _D = 4, 8192, 4096
_N = _B * _S
_NC, _NS = 2, 16
_NW = _NC * _NS
_PER_W = _N // _NW
_ROWS = 128
_NBOX = _PER_W // _ROWS
_L = 16
_IDX = 3

_mesh = plsc.VectorSubcoreMesh(core_axis_name="c", subcore_axis_name="s")


@functools.partial(
    pl.kernel,
    mesh=_mesh,
    out_type=jax.ShapeDtypeStruct((_NW, _NBOX, 128), jnp.float32),
    scratch_types=[
        pltpu.VMEM((2, _ROWS, 128), jnp.float32),
        pltpu.VMEM((_NBOX, 128), jnp.float32),
        pltpu.SemaphoreType.DMA,
        pltpu.SemaphoreType.DMA,
    ],
    compiler_params=pltpu.CompilerParams(
        use_tc_tiling_on_sc=True, needs_layout_passes=False
    ),
)
def _select_copy(in_hbm, out_hbm, tiles_v, out_v, sem0, sem1):
    wid = lax.axis_index("s") * _NC + lax.axis_index("c")
    base = wid * _PER_W
    sems = (sem0, sem1)
    copies = [
        pltpu.make_async_copy(
            in_hbm.at[pl.ds(base + c * _ROWS, _ROWS), pl.ds(0, 128)],
            tiles_v.at[c % 2],
            sems[c % 2],
        )
        for c in range(_NBOX)
    ]
    copies[0].start()
    copies[1].start()
    iota = lax.iota(jnp.int32, _L)
    lane = jnp.full((_L,), _IDX, dtype=jnp.int32)
    for c in range(_NBOX):
        copies[c].wait()
        crow = jnp.full((_L,), c, dtype=jnp.int32)
        for j in range(_ROWS // _L):
            rows = iota + j * _L
            vals = plsc.load_gather(tiles_v.at[c % 2], [rows, lane])
            plsc.store_scatter(out_v, [crow, iota + j * _L], vals)
        if c + 2 < _NBOX:
            copies[c + 2].start()
    pltpu.sync_copy(out_v, out_hbm.at[wid])


def kernel(input_):
    in2d = input_.reshape(_N, _D)
    out = _select_copy(in2d)
    return out.reshape(_B, _S, 1)
